# baseline (speedup 1.0000x reference)
import functools

import jax
import jax.numpy as jnp
from jax import lax
from jax.experimental import pallas as pl
from jax.experimental.pallas import tpu as pltpu
from jax.experimental.pallas import tpu_sc as plsc

B = 128
N = 8192
L = 16
NVEC = N // L


def _splat(x):
    return jnp.full((L,), x, jnp.float32)


def _isplat(x):
    return jnp.full((L,), x, jnp.int32)


def _compress_body(v, m, off, sacc, dst):
    sacc = sacc + jnp.where(m, v, jnp.float32(0.0))
    cum = plsc.cumsum(m.astype(jnp.int32))
    plsc.store_scatter(dst, [off + cum - 1], v, mask=m)
    off = off + plsc.all_reduce_population_count(m)
    return off, sacc


def _first_compress(xv, r, t, dst):
    def body(i, carry):
        off, sacc = carry
        v = xv[r, pl.ds(i * L, L)]
        m = v > t
        return _compress_body(v, m, off, sacc, dst)

    off, sacc = lax.fori_loop(
        0, NVEC, body, (jnp.zeros((L,), jnp.int32), jnp.zeros((L,), jnp.float32))
    )
    return off, _splat(jnp.sum(sacc))


def _compress(src, klen, k_splat, t, dst):
    nv = lax.shift_right_logical(klen + (L - 1), 4)
    lanes = lax.iota(jnp.int32, L)

    def body(i, carry):
        off, sacc = carry
        v = src[pl.ds(i * L, L)]
        valid = (lanes + _isplat(i * L)) < k_splat
        m = valid & (v > t)
        return _compress_body(v, m, off, sacc, dst)

    off, sacc = lax.fori_loop(
        0, nv, body, (jnp.zeros((L,), jnp.int32), jnp.zeros((L,), jnp.float32))
    )
    return off, _splat(jnp.sum(sacc))


def _row_fixpoint(xv, r, buf0, buf1):
    def sum_body(i, accs):
        base = i * (4 * L)
        return tuple(
            acc + xv[r, pl.ds(base + j * L, L)] for j, acc in enumerate(accs)
        )

    zero = jnp.zeros((L,), jnp.float32)
    accs0 = lax.fori_loop(0, NVEC // 4, sum_body, (zero, zero, zero, zero))
    acc0 = (accs0[0] + accs0[1]) + (accs0[2] + accs0[3])
    total = _splat(jnp.sum(acc0))
    t0 = (total - 1.0) / _splat(jnp.float32(N))

    k1, s1 = _first_compress(xv, r, t0, buf0)
    t1 = (s1 - 1.0) / k1.astype(jnp.float32)
    klen1 = jnp.max(k1)

    def cond(carry):
        _, k, prev_k, _ = carry
        return jnp.any(k != prev_k)

    def body(carry):
        t, k, _, klen = carry
        k2, s2 = _compress(buf0, klen, k, t, buf1)
        t2 = (s2 - 1.0) / k2.astype(jnp.float32)
        klen2 = jnp.max(k2)
        k3, s3 = _compress(buf1, klen2, k2, t2, buf0)
        t3 = (s3 - 1.0) / k3.astype(jnp.float32)
        klen3 = jnp.max(k3)
        return t3, k3, k, klen3

    t, k, _, _ = lax.while_loop(cond, body, (t1, k1, _isplat(N), klen1))
    return t - 1.0 / k.astype(jnp.float32)


def _make_sc_kernel():
    info = plsc.get_sparse_core_info()
    nw = info.num_cores * info.num_subcores
    rows_per_w = B // nw
    mesh = plsc.VectorSubcoreMesh(core_axis_name="c", subcore_axis_name="s")

    @functools.partial(
        pl.kernel,
        mesh=mesh,
        out_type=jax.ShapeDtypeStruct((B, N), jnp.float32),
        scratch_types=[
            pltpu.VMEM((rows_per_w, N), jnp.float32),
            pltpu.VMEM((N,), jnp.float32),
            pltpu.VMEM((N,), jnp.float32),
        ],
        compiler_params=pltpu.CompilerParams(needs_layout_passes=False),
    )
    def sparsemax_sc(x_hbm, out_hbm, xv, buf0, buf1):
        wid = lax.axis_index("s") * info.num_cores + lax.axis_index("c")
        base = wid * rows_per_w
        pltpu.sync_copy(x_hbm.at[pl.ds(base, rows_per_w)], xv)
        for r in range(rows_per_w):
            tau = _row_fixpoint(xv, r, buf0, buf1)

            def out_body(i, _, r=r, tau=tau):
                base = i * (4 * L)
                for j in range(4):
                    v = xv[r, pl.ds(base + j * L, L)]
                    xv[r, pl.ds(base + j * L, L)] = jnp.maximum(
                        v - tau, jnp.float32(0.0)
                    )
                return 0

            lax.fori_loop(0, NVEC // 4, out_body, 0)
        pltpu.sync_copy(xv, out_hbm.at[pl.ds(base, rows_per_w)])

    return sparsemax_sc


_sparsemax = _make_sc_kernel()


def kernel(input):
    return _sparsemax(input)


# speedup vs baseline: 1.4109x; 1.4109x over previous
import functools

import jax
import jax.numpy as jnp
from jax import lax
from jax.experimental import pallas as pl
from jax.experimental.pallas import tpu as pltpu
from jax.experimental.pallas import tpu_sc as plsc

B = 128
N = 8192
L = 16
NVEC = N // L
NEG_INF = jnp.float32(-jnp.inf)


def _splat(x):
    return jnp.full((L,), x, jnp.float32)


def _append_step(v, t, dst, off, sacc, kacc):
    m = v > t
    sacc = sacc + jnp.where(m, v, jnp.float32(0.0))
    pop = plsc.all_reduce_population_count(m)
    kacc = kacc + pop
    dst[pl.ds(off, L)] = jnp.where(m, v, NEG_INF)
    off = off + jnp.where(jnp.any(m), jnp.int32(L), jnp.int32(0))
    return off, sacc, kacc


def _first_pass(xv, r, t, dst):
    def body(i, carry):
        off, sacc, kacc = carry
        base = i * (4 * L)
        for j in range(4):
            v = xv[r, pl.ds(base + j * L, L)]
            off, sacc, kacc = _append_step(v, t, dst, off, sacc, kacc)
        return off, sacc, kacc

    off, sacc, kacc = lax.fori_loop(
        0,
        NVEC // 4,
        body,
        (jnp.int32(0), jnp.zeros((L,), jnp.float32), jnp.zeros((L,), jnp.int32)),
    )
    return off, kacc, _splat(jnp.sum(sacc))


def _compact_pass(src, klen, t, dst):
    nv = lax.shift_right_logical(klen, 4)

    def body(i, carry):
        off, sacc, kacc = carry
        v = src[pl.ds(i * L, L)]
        off, sacc, kacc = _append_step(v, t, dst, off, sacc, kacc)
        return off, sacc, kacc

    off, sacc, kacc = lax.fori_loop(
        0,
        nv,
        body,
        (jnp.int32(0), jnp.zeros((L,), jnp.float32), jnp.zeros((L,), jnp.int32)),
    )
    return off, kacc, _splat(jnp.sum(sacc))


def _row_fixpoint(xv, r, buf0, buf1):
    def sum_max_body(i, carry):
        s0, s1, s2, s3, m0, m1 = carry
        base = i * (4 * L)
        v0 = xv[r, pl.ds(base + 0 * L, L)]
        v1 = xv[r, pl.ds(base + 1 * L, L)]
        v2 = xv[r, pl.ds(base + 2 * L, L)]
        v3 = xv[r, pl.ds(base + 3 * L, L)]
        return (
            s0 + v0,
            s1 + v1,
            s2 + v2,
            s3 + v3,
            jnp.maximum(m0, jnp.maximum(v0, v1)),
            jnp.maximum(m1, jnp.maximum(v2, v3)),
        )

    zero = jnp.zeros((L,), jnp.float32)
    ninf = jnp.full((L,), NEG_INF)
    s0, s1, s2, s3, m0, m1 = lax.fori_loop(
        0, NVEC // 4, sum_max_body, (zero, zero, zero, zero, ninf, ninf)
    )
    total = _splat(jnp.sum((s0 + s1) + (s2 + s3)))
    zmax = _splat(jnp.max(jnp.maximum(m0, m1)))
    t0 = (total - 1.0) / _splat(jnp.float32(N))
    t_start = jnp.maximum(t0, zmax - 1.0)

    klen1, k1, s1v = _first_pass(xv, r, t_start, buf0)
    t1 = (s1v - 1.0) / k1.astype(jnp.float32)

    def cond(carry):
        _, k, prev_k, _ = carry
        return jnp.any(k != prev_k)

    def body(carry):
        t, k, _, klen = carry
        klen2, k2, s2v = _compact_pass(buf0, klen, t, buf1)
        t2 = (s2v - 1.0) / k2.astype(jnp.float32)
        klen3, k3, s3v = _compact_pass(buf1, klen2, t2, buf0)
        t3 = (s3v - 1.0) / k3.astype(jnp.float32)
        return t3, k3, k, klen3

    t, k, _, _ = lax.while_loop(
        cond, body, (t1, k1, jnp.full((L,), -1, jnp.int32), klen1)
    )
    return t - 1.0 / k.astype(jnp.float32)


def _make_sc_kernel():
    info = plsc.get_sparse_core_info()
    nw = info.num_cores * info.num_subcores
    rows_per_w = B // nw
    mesh = plsc.VectorSubcoreMesh(core_axis_name="c", subcore_axis_name="s")

    @functools.partial(
        pl.kernel,
        mesh=mesh,
        out_type=jax.ShapeDtypeStruct((B, N), jnp.float32),
        scratch_types=[
            pltpu.VMEM((rows_per_w, N), jnp.float32),
            pltpu.VMEM((N,), jnp.float32),
            pltpu.VMEM((N,), jnp.float32),
        ],
        compiler_params=pltpu.CompilerParams(needs_layout_passes=False),
    )
    def sparsemax_sc(x_hbm, out_hbm, xv, buf0, buf1):
        wid = lax.axis_index("s") * info.num_cores + lax.axis_index("c")
        base = wid * rows_per_w
        pltpu.sync_copy(x_hbm.at[pl.ds(base, rows_per_w)], xv)
        for r in range(rows_per_w):
            tau = _row_fixpoint(xv, r, buf0, buf1)

            def out_body(i, _, r=r, tau=tau):
                base = i * (4 * L)
                for j in range(4):
                    v = xv[r, pl.ds(base + j * L, L)]
                    xv[r, pl.ds(base + j * L, L)] = jnp.maximum(
                        v - tau, jnp.float32(0.0)
                    )
                return 0

            lax.fori_loop(0, NVEC // 4, out_body, 0)
        pltpu.sync_copy(xv, out_hbm.at[pl.ds(base, rows_per_w)])

    return sparsemax_sc


_sparsemax = _make_sc_kernel()


def kernel(input):
    return _sparsemax(input)


# speedup vs baseline: 1.8499x; 1.3112x over previous
import functools

import jax
import jax.numpy as jnp
from jax import lax
from jax.experimental import pallas as pl
from jax.experimental.pallas import tpu as pltpu
from jax.experimental.pallas import tpu_sc as plsc

B = 128
N = 8192
L = 16
NVEC = N // L
NEG_INF = float("-inf")


def _splat(x):
    return jnp.full((L,), x, jnp.float32)


def _append_step(v, t, dst, off, sacc, kacc, iota):
    m = v > t
    sacc = sacc + jnp.where(m, v, jnp.float32(0.0))
    pop = plsc.all_reduce_population_count(m)
    kacc = kacc + pop
    plsc.store_scatter(dst, [off + iota], jnp.where(m, v, jnp.float32(NEG_INF)))
    off = off + jnp.where(pop > 0, jnp.int32(L), jnp.int32(0))
    return off, sacc, kacc


def _first_pass(xv, r, t, dst):
    iota = lax.iota(jnp.int32, L)

    def body(i, carry):
        off, sacc, kacc = carry
        base = i * (4 * L)
        for j in range(4):
            v = xv[r, pl.ds(base + j * L, L)]
            off, sacc, kacc = _append_step(v, t, dst, off, sacc, kacc, iota)
        return off, sacc, kacc

    off, sacc, kacc = lax.fori_loop(
        0,
        NVEC // 4,
        body,
        (
            jnp.zeros((L,), jnp.int32),
            jnp.zeros((L,), jnp.float32),
            jnp.zeros((L,), jnp.int32),
        ),
    )
    return jnp.max(off), kacc, _splat(jnp.sum(sacc))


def _compact_pass(src, klen, t, dst):
    nv = lax.shift_right_logical(klen, 4)
    iota = lax.iota(jnp.int32, L)

    def body(i, carry):
        off, sacc, kacc = carry
        v = src[pl.ds(i * L, L)]
        off, sacc, kacc = _append_step(v, t, dst, off, sacc, kacc, iota)
        return off, sacc, kacc

    off, sacc, kacc = lax.fori_loop(
        0,
        nv,
        body,
        (
            jnp.zeros((L,), jnp.int32),
            jnp.zeros((L,), jnp.float32),
            jnp.zeros((L,), jnp.int32),
        ),
    )
    return jnp.max(off), kacc, _splat(jnp.sum(sacc))


def _row_fixpoint(xv, r, buf0, buf1):
    def max_body(i, carry):
        base = i * (8 * L)
        vs = [xv[r, pl.ds(base + j * L, L)] for j in range(8)]
        return tuple(
            jnp.maximum(carry[j], jnp.maximum(vs[2 * j], vs[2 * j + 1]))
            for j in range(4)
        )

    ninf = jnp.full((L,), NEG_INF, jnp.float32)
    m0, m1, m2, m3 = lax.fori_loop(0, NVEC // 8, max_body, (ninf,) * 4)
    zmax = _splat(jnp.max(jnp.maximum(jnp.maximum(m0, m1), jnp.maximum(m2, m3))))
    t_start = zmax - 1.0

    klen1, k1, s1v = _first_pass(xv, r, t_start, buf0)
    t1 = (s1v - 1.0) / k1.astype(jnp.float32)

    def cond(carry):
        _, k, prev_k, _ = carry
        return jnp.any(k != prev_k)

    def body(carry):
        t, k, _, klen = carry
        klen2, k2, s2v = _compact_pass(buf0, klen, t, buf1)
        t2 = (s2v - 1.0) / k2.astype(jnp.float32)
        klen3, k3, s3v = _compact_pass(buf1, klen2, t2, buf0)
        t3 = (s3v - 1.0) / k3.astype(jnp.float32)
        return t3, k3, k, klen3

    t, k, _, _ = lax.while_loop(
        cond, body, (t1, k1, jnp.full((L,), -1, jnp.int32), klen1)
    )
    return t - 1.0 / k.astype(jnp.float32)


def _make_sc_kernel():
    info = plsc.get_sparse_core_info()
    nw = info.num_cores * info.num_subcores
    rows_per_w = B // nw
    mesh = plsc.VectorSubcoreMesh(core_axis_name="c", subcore_axis_name="s")

    @functools.partial(
        pl.kernel,
        mesh=mesh,
        out_type=jax.ShapeDtypeStruct((B, N), jnp.float32),
        scratch_types=[
            pltpu.VMEM((rows_per_w, N), jnp.float32),
            pltpu.VMEM((N,), jnp.float32),
            pltpu.VMEM((N,), jnp.float32),
        ],
        compiler_params=pltpu.CompilerParams(needs_layout_passes=False),
    )
    def sparsemax_sc(x_hbm, out_hbm, xv, buf0, buf1):
        wid = lax.axis_index("s") * info.num_cores + lax.axis_index("c")
        base = wid * rows_per_w
        pltpu.sync_copy(x_hbm.at[pl.ds(base, rows_per_w)], xv)
        for r in range(rows_per_w):
            tau = _row_fixpoint(xv, r, buf0, buf1)

            def out_body(i, _, r=r, tau=tau):
                base = i * (4 * L)
                for j in range(4):
                    v = xv[r, pl.ds(base + j * L, L)]
                    xv[r, pl.ds(base + j * L, L)] = jnp.maximum(
                        v - tau, jnp.float32(0.0)
                    )
                return 0

            lax.fori_loop(0, NVEC // 4, out_body, 0)
        pltpu.sync_copy(xv, out_hbm.at[pl.ds(base, rows_per_w)])

    return sparsemax_sc


_sparsemax = _make_sc_kernel()


def kernel(input):
    return _sparsemax(input)


# speedup vs baseline: 2.0503x; 1.1084x over previous
import functools

import jax
import jax.numpy as jnp
from jax import lax
from jax.experimental import pallas as pl
from jax.experimental.pallas import tpu as pltpu
from jax.experimental.pallas import tpu_sc as plsc

B = 128
N = 8192
L = 16
NVEC = N // L
NEG_INF = float("-inf")


def _splat(x):
    return jnp.full((L,), x, jnp.float32)


def _append_step(v, t, dst, off, sacc, kacc, iota):
    m = v > t
    sacc = sacc + jnp.where(m, v, jnp.float32(0.0))
    pop = plsc.all_reduce_population_count(m)
    kacc = kacc + pop
    plsc.store_scatter(dst, [off + iota], jnp.where(m, v, jnp.float32(NEG_INF)))
    off = off + jnp.where(pop > 0, jnp.int32(L), jnp.int32(0))
    return off, sacc, kacc


def _first_pass(xv, r, t, dst):
    iota = lax.iota(jnp.int32, L)

    def body(i, carry):
        off, sacc, kacc = carry
        base = i * (8 * L)
        for j in range(8):
            v = xv[r, pl.ds(base + j * L, L)]
            off, sacc, kacc = _append_step(v, t, dst, off, sacc, kacc, iota)
        return off, sacc, kacc

    off, sacc, kacc = lax.fori_loop(
        0,
        NVEC // 8,
        body,
        (
            jnp.zeros((L,), jnp.int32),
            jnp.zeros((L,), jnp.float32),
            jnp.zeros((L,), jnp.int32),
        ),
    )
    return jnp.max(off), kacc, _splat(jnp.sum(sacc))


def _compact_pass(src, klen, t, dst):
    nv = lax.shift_right_logical(klen, 4)
    iota = lax.iota(jnp.int32, L)

    def body(i, carry):
        off, sacc, kacc = carry
        v = src[pl.ds(i * L, L)]
        off, sacc, kacc = _append_step(v, t, dst, off, sacc, kacc, iota)
        return off, sacc, kacc

    off, sacc, kacc = lax.fori_loop(
        0,
        nv,
        body,
        (
            jnp.zeros((L,), jnp.int32),
            jnp.zeros((L,), jnp.float32),
            jnp.zeros((L,), jnp.int32),
        ),
    )
    return jnp.max(off), kacc, _splat(jnp.sum(sacc))


def _row_fixpoint(xv, r, buf0, buf1):
    ninf = jnp.full((L,), NEG_INF, jnp.float32)

    @plsc.parallel_loop(0, N, step=8 * L, unroll=2, carry=(ninf,) * 4)
    def max_accs(base, carry):
        vs = [xv[r, pl.ds(base + j * L, L)] for j in range(8)]
        return tuple(
            jnp.maximum(carry[j], jnp.maximum(vs[2 * j], vs[2 * j + 1]))
            for j in range(4)
        )

    m0, m1, m2, m3 = max_accs
    zmax = _splat(jnp.max(jnp.maximum(jnp.maximum(m0, m1), jnp.maximum(m2, m3))))
    t_start = zmax - 1.0

    klen1, k1, s1v = _first_pass(xv, r, t_start, buf0)
    t1 = (s1v - 1.0) / k1.astype(jnp.float32)

    def cond(carry):
        _, k, prev_k, _ = carry
        return jnp.any(k != prev_k) & jnp.any(k > L)

    def body(carry):
        t, k, _, klen = carry
        klen2, k2, s2v = _compact_pass(buf0, klen, t, buf1)
        t2 = (s2v - 1.0) / k2.astype(jnp.float32)
        klen3, k3, s3v = _compact_pass(buf1, klen2, t2, buf0)
        t3 = (s3v - 1.0) / k3.astype(jnp.float32)
        return t3, k3, k, klen3

    t, k, _, klen = lax.while_loop(
        cond, body, (t1, k1, jnp.full((L,), -1, jnp.int32), klen1)
    )
    tau_fix = t - 1.0 / k.astype(jnp.float32)

    iota = lax.iota(jnp.int32, L)
    buf1[pl.ds(0, L)] = jnp.full((L,), NEG_INF, jnp.float32)

    def consolidate(i, off):
        v = buf0[pl.ds(i * L, L)]
        m = v > jnp.float32(NEG_INF)
        cum = plsc.cumsum(m.astype(jnp.int32))
        plsc.store_scatter(buf1, [off + cum - 1], v, mask=m)
        return off + plsc.all_reduce_population_count(m)

    nv = lax.shift_right_logical(klen, 4)
    lax.fori_loop(0, nv, consolidate, jnp.zeros((L,), jnp.int32))
    z, _ = plsc.sort_key_val(buf1[pl.ds(0, L)], iota, descending=True)
    cum = plsc.cumsum(z)
    idxf = (iota + 1).astype(jnp.float32)
    valid = (idxf * z) > (cum - 1.0)
    rho = plsc.all_reduce_population_count(valid).astype(jnp.float32)
    s_rho = _splat(jnp.sum(jnp.where(valid, z, jnp.float32(0.0))))
    tau_sort = (s_rho - 2.0) / rho

    return jnp.where(k <= L, tau_sort, tau_fix)


def _make_sc_kernel():
    info = plsc.get_sparse_core_info()
    nw = info.num_cores * info.num_subcores
    rows_per_w = B // nw
    mesh = plsc.VectorSubcoreMesh(core_axis_name="c", subcore_axis_name="s")

    @functools.partial(
        pl.kernel,
        mesh=mesh,
        out_type=jax.ShapeDtypeStruct((B, N), jnp.float32),
        scratch_types=[
            pltpu.VMEM((rows_per_w, N), jnp.float32),
            pltpu.VMEM((N,), jnp.float32),
            pltpu.VMEM((N,), jnp.float32),
        ],
        compiler_params=pltpu.CompilerParams(needs_layout_passes=False),
    )
    def sparsemax_sc(x_hbm, out_hbm, xv, buf0, buf1):
        wid = lax.axis_index("s") * info.num_cores + lax.axis_index("c")
        base = wid * rows_per_w
        pltpu.sync_copy(x_hbm.at[pl.ds(base, rows_per_w)], xv)
        for r in range(rows_per_w):
            tau = _row_fixpoint(xv, r, buf0, buf1)

            @plsc.parallel_loop(0, N, step=8 * L, unroll=2)
            def out_body(base, r=r, tau=tau):
                for j in range(8):
                    v = xv[r, pl.ds(base + j * L, L)]
                    xv[r, pl.ds(base + j * L, L)] = jnp.maximum(
                        v - tau, jnp.float32(0.0)
                    )
        pltpu.sync_copy(xv, out_hbm.at[pl.ds(base, rows_per_w)])

    return sparsemax_sc


_sparsemax = _make_sc_kernel()


def kernel(input):
    return _sparsemax(input)


# speedup vs baseline: 2.0567x; 1.0031x over previous
import functools

import jax
import jax.numpy as jnp
from jax import lax
from jax.experimental import pallas as pl
from jax.experimental.pallas import tpu as pltpu
from jax.experimental.pallas import tpu_sc as plsc

B = 128
N = 8192
L = 16
NVEC = N // L
NEG_INF = float("-inf")


def _splat(x):
    return jnp.full((L,), x, jnp.float32)


def _first_pass(xv, r, t, dst):
    iota = lax.iota(jnp.int32, L)

    def body(i, off):
        base = i * (4 * L)
        for j in range(4):
            v = xv[r, pl.ds(base + j * L, L)]
            m = v > t
            pop = plsc.all_reduce_population_count(m)
            plsc.store_scatter(
                dst, [off + iota], jnp.where(m, v, jnp.float32(NEG_INF))
            )
            off = off + jnp.where(pop > 0, jnp.int32(L), jnp.int32(0))
        return off

    off = lax.fori_loop(0, NVEC // 4, body, jnp.zeros((L,), jnp.int32))
    return jnp.max(off)


def _compact_pass(src, klen, t, dst):
    nv = lax.shift_right_logical(klen, 4)
    iota = lax.iota(jnp.int32, L)

    def body(i, carry):
        off, sacc, kacc = carry
        v = src[pl.ds(i * L, L)]
        m = v > t
        sacc = sacc + jnp.where(m, v, jnp.float32(0.0))
        pop = plsc.all_reduce_population_count(m)
        kacc = kacc + pop
        plsc.store_scatter(dst, [off + iota], jnp.where(m, v, jnp.float32(NEG_INF)))
        off = off + jnp.where(pop > 0, jnp.int32(L), jnp.int32(0))
        return off, sacc, kacc

    off, sacc, kacc = lax.fori_loop(
        0,
        nv,
        body,
        (
            jnp.zeros((L,), jnp.int32),
            jnp.zeros((L,), jnp.float32),
            jnp.zeros((L,), jnp.int32),
        ),
    )
    return jnp.max(off), kacc, _splat(jnp.sum(sacc))


def _row_fixpoint(xv, r, buf0, buf1):
    ninf = jnp.full((L,), NEG_INF, jnp.float32)

    @plsc.parallel_loop(0, N, step=4 * L, unroll=2, carry=(ninf,) * 2)
    def max_accs(base, carry):
        vs = [xv[r, pl.ds(base + j * L, L)] for j in range(4)]
        return tuple(
            jnp.maximum(carry[j], jnp.maximum(vs[2 * j], vs[2 * j + 1]))
            for j in range(2)
        )

    m0, m1 = max_accs
    zmax = _splat(jnp.max(jnp.maximum(m0, m1)))
    t_start = zmax - 1.0

    klen1 = _first_pass(xv, r, t_start, buf0)

    def cond(carry):
        _, k, prev_k, _ = carry
        return jnp.any(k != prev_k) & jnp.any(k > L)

    def body(carry):
        t, k, _, klen = carry
        klen2, k2, s2v = _compact_pass(buf0, klen, t, buf1)
        t2 = (s2v - 1.0) / k2.astype(jnp.float32)
        klen3, k3, s3v = _compact_pass(buf1, klen2, t2, buf0)
        t3 = (s3v - 1.0) / k3.astype(jnp.float32)
        return t3, k3, k2, klen3

    t, k, _, klen = lax.while_loop(
        cond,
        body,
        (t_start, jnp.full((L,), N, jnp.int32), jnp.full((L,), -1, jnp.int32), klen1),
    )
    tau_fix = t - 1.0 / k.astype(jnp.float32)

    iota = lax.iota(jnp.int32, L)
    buf1[pl.ds(0, L)] = jnp.full((L,), NEG_INF, jnp.float32)

    def consolidate(i, off):
        v = buf0[pl.ds(i * L, L)]
        m = v > jnp.float32(NEG_INF)
        cum = plsc.cumsum(m.astype(jnp.int32))
        plsc.store_scatter(buf1, [off + cum - 1], v, mask=m)
        return off + plsc.all_reduce_population_count(m)

    nv = lax.shift_right_logical(klen, 4)
    lax.fori_loop(0, nv, consolidate, jnp.zeros((L,), jnp.int32))
    z, _ = plsc.sort_key_val(buf1[pl.ds(0, L)], iota, descending=True)
    cum = plsc.cumsum(z)
    idxf = (iota + 1).astype(jnp.float32)
    valid = (idxf * z) > (cum - 1.0)
    rho = plsc.all_reduce_population_count(valid).astype(jnp.float32)
    s_rho = _splat(jnp.sum(jnp.where(valid, z, jnp.float32(0.0))))
    tau_sort = (s_rho - 2.0) / rho

    return jnp.where(k <= L, tau_sort, tau_fix)


def _make_sc_kernel():
    info = plsc.get_sparse_core_info()
    nw = info.num_cores * info.num_subcores
    rows_per_w = B // nw
    mesh = plsc.VectorSubcoreMesh(core_axis_name="c", subcore_axis_name="s")

    @functools.partial(
        pl.kernel,
        mesh=mesh,
        out_type=jax.ShapeDtypeStruct((B, N), jnp.float32),
        scratch_types=[
            pltpu.VMEM((rows_per_w, N), jnp.float32),
            pltpu.VMEM((N,), jnp.float32),
            pltpu.VMEM((N,), jnp.float32),
        ],
        compiler_params=pltpu.CompilerParams(needs_layout_passes=False),
    )
    def sparsemax_sc(x_hbm, out_hbm, xv, buf0, buf1):
        wid = lax.axis_index("s") * info.num_cores + lax.axis_index("c")
        base = wid * rows_per_w
        pltpu.sync_copy(x_hbm.at[pl.ds(base, rows_per_w)], xv)
        for r in range(rows_per_w):
            tau = _row_fixpoint(xv, r, buf0, buf1)

            @plsc.parallel_loop(0, N, step=4 * L, unroll=2)
            def out_body(base, r=r, tau=tau):
                for j in range(4):
                    v = xv[r, pl.ds(base + j * L, L)]
                    xv[r, pl.ds(base + j * L, L)] = jnp.maximum(
                        v - tau, jnp.float32(0.0)
                    )
        pltpu.sync_copy(xv, out_hbm.at[pl.ds(base, rows_per_w)])

    return sparsemax_sc


_sparsemax = _make_sc_kernel()


def kernel(input):
    return _sparsemax(input)
